# baseline (speedup 1.0000x reference)
import functools

import jax
import jax.numpy as jnp
from jax import lax
from jax.experimental import pallas as pl
from jax.experimental.pallas import tpu as pltpu
from jax.experimental.pallas import tpu_sc as plsc

_E = 320000
_D = 128
_N = 10000
_HN = _N // 2
_NC = 2
_NS = 16
_CH = 128
_NCHT = _E // _CH
_NB = 4
_AN = _HN + 8
_ZPT = _AN // _NS
_OPT = 312
_SPT = _E // _NS


def _sc_body(x_hbm, idx_hbm, out_hbm, idx_v, x_v, scan_v, cx_v, cxr_v, acc,
             cnts_sh, sem_in, sem_sc):
    cid = lax.axis_index("c")
    sid = lax.axis_index("s")

    pltpu.sync_copy(idx_hbm.at[pl.ds(sid * _SPT, _SPT)], scan_v.at[pl.ds(0, _SPT)])

    def _bstep(_, lohi):
        blo, bhi = lohi
        mid = (blo + bhi) // 2
        v = scan_v[pl.ds(mid, 16)]
        open_ = blo < bhi
        pred = jnp.logical_and(open_, v[0] < _HN)
        return (jnp.where(pred, mid + 1, blo),
                jnp.where(jnp.logical_and(open_, jnp.logical_not(pred)), mid, bhi))

    cnt, _ = lax.fori_loop(
        0, 15, _bstep, (jnp.int32(0), jnp.int32(_SPT)))
    cx_v[...] = jnp.full((16,), cnt, jnp.int32)
    pltpu.sync_copy(cx_v, cnts_sh.at[pl.ds(sid * 16, 16)])

    zero16 = jnp.zeros((16,), jnp.float32)

    def _zrow(r, c):
        def _zcol(j, cc):
            x_v[_NB - 1, r, pl.ds(j * 16, 16)] = zero16
            return cc
        return lax.fori_loop(0, _D // 16, _zcol, c)

    lax.fori_loop(0, _CH, _zrow, 0)

    for k in range(_ZPT // _CH):
        pltpu.sync_copy(x_v.at[_NB - 1], acc.at[pl.ds(sid * _ZPT + k * _CH, _CH), :])
    _zt = _ZPT % _CH
    pltpu.sync_copy(
        x_v.at[_NB - 1, pl.ds(0, _zt), :],
        acc.at[pl.ds(sid * _ZPT + _ZPT - _zt, _zt), :],
    )
    plsc.subcore_barrier()

    pltpu.sync_copy(cnts_sh, cxr_v)
    svec = cxr_v[pl.ds(0, 16)]
    for r in range(1, _NS):
        svec = svec + cxr_v[pl.ds(r * 16, 16)]
    s = jnp.minimum(jnp.maximum(svec[0], 0), _E)

    c0 = (s + _CH - 1) // _CH
    c1 = s // _CH
    m = jnp.where(cid == 0, c0, _NCHT - c1)
    base_c = jnp.where(cid == 0, 0, c1)
    lo = base_c + (m * sid) // _NS
    hi = base_c + (m * (sid + 1)) // _NS
    kstop = hi - lo

    def _start_load(c, b):
        pos = jnp.maximum(lo + jnp.minimum(c, kstop - 1), 0)
        pltpu.async_copy(
            idx_hbm.at[pl.ds(pos * _CH, _CH)], idx_v.at[b], sem_in.at[b])
        pltpu.async_copy(
            x_hbm.at[pl.ds(pos * _CH, _CH), :], x_v.at[b], sem_in.at[b])

    def _wait_load(b):
        pltpu.make_async_copy(
            idx_hbm.at[pl.ds(0, _CH)], idx_v.at[b], sem_in.at[b]).wait()
        pltpu.make_async_copy(
            x_hbm.at[pl.ds(0, _CH), :], x_v.at[b], sem_in.at[b]).wait()

    def _scatter_desc(b):
        return pltpu.make_async_copy(x_v.at[b], acc.at[idx_v.at[b]], sem_sc.at[b])

    base_row = cid * _HN

    def _consume(c, b):
        _wait_load(b)
        voff = jnp.full(
            (16,), base_row - jnp.where(c < kstop, 0, 2 * _HN), jnp.int32)
        for q in range(_CH // 16):
            v = idx_v[b, pl.ds(q * 16, 16)]
            loc = jnp.minimum(
                (v - voff).astype(jnp.uint32), jnp.uint32(_HN))
            idx_v[b, pl.ds(q * 16, 16)] = loc.astype(jnp.int32)
        pltpu.async_copy(x_v.at[b], acc.at[idx_v.at[b]], sem_sc.at[b], add=True)

    for b in range(_NB - 1):
        _start_load(b, b)

    for b in range(_NB):
        bp = (b + _NB - 1) % _NB
        if b >= 1:
            _scatter_desc(bp).wait()
        _start_load(b + _NB - 1, bp)
        _consume(b, b)

    def _group(j, carry):
        for b in range(_NB):
            c = j * _NB + b
            bp = (b + _NB - 1) % _NB
            _scatter_desc(bp).wait()
            _start_load(c + _NB - 1, bp)
            _consume(c, b)
        return carry

    n_groups = jnp.maximum((kstop + _NB - 1) // _NB, 1)
    lax.fori_loop(1, n_groups, _group, 0)

    _scatter_desc(_NB - 1).wait()
    for b in range(_NB - 1):
        _wait_load(b)
    plsc.subcore_barrier()

    pltpu.sync_copy(
        acc.at[pl.ds(sid * _OPT, _OPT), :],
        out_hbm.at[pl.ds(base_row + sid * _OPT, _OPT), :],
    )

    @pl.when(sid == _NS - 1)
    def _tail():
        pltpu.sync_copy(
            acc.at[pl.ds(_NS * _OPT, _HN - _NS * _OPT), :],
            out_hbm.at[pl.ds(base_row + _NS * _OPT, _HN - _NS * _OPT), :],
        )


_sc_scatter = functools.partial(
    pl.kernel,
    out_type=jax.ShapeDtypeStruct((_N, _D), jnp.float32),
    mesh=plsc.VectorSubcoreMesh(core_axis_name="c", subcore_axis_name="s"),
    scratch_types=[
        pltpu.VMEM((_NB, _CH), jnp.int32),
        pltpu.VMEM((_NB, _CH, _D), jnp.float32),
        pltpu.VMEM((_SPT + 16,), jnp.int32),
        pltpu.VMEM((16,), jnp.int32),
        pltpu.VMEM((_NS * 16,), jnp.int32),
        pltpu.VMEM_SHARED((_AN, _D), jnp.float32),
        pltpu.VMEM_SHARED((_NS * 16,), jnp.int32),
        pltpu.SemaphoreType.DMA((_NB,)),
        pltpu.SemaphoreType.DMA((_NB,)),
    ],
)(_sc_body)


def kernel(x, index, dim_size):
    del dim_size
    return _sc_scatter(x, index.astype(jnp.int32))


# speedup vs baseline: 1.0058x; 1.0058x over previous
import functools

import jax
import jax.numpy as jnp
from jax import lax
from jax.experimental import pallas as pl
from jax.experimental.pallas import tpu as pltpu
from jax.experimental.pallas import tpu_sc as plsc

_E = 320000
_D = 128
_N = 10000
_HN = _N // 2
_NC = 2
_NS = 16
_CH = 64
_NCHT = _E // _CH
_NB = 8
_AN = _HN + 8
_ZPT = _AN // _NS
_OPT = 312
_SPT = _E // _NS


def _sc_body(x_hbm, idx_hbm, out_hbm, idx_v, x_v, scan_v, cx_v, cxr_v, acc,
             cnts_sh, sem_in, sem_sc):
    cid = lax.axis_index("c")
    sid = lax.axis_index("s")

    pltpu.sync_copy(idx_hbm.at[pl.ds(sid * _SPT, _SPT)], scan_v.at[pl.ds(0, _SPT)])

    def _bstep(_, lohi):
        blo, bhi = lohi
        mid = (blo + bhi) // 2
        v = scan_v[pl.ds(mid, 16)]
        open_ = blo < bhi
        pred = jnp.logical_and(open_, v[0] < _HN)
        return (jnp.where(pred, mid + 1, blo),
                jnp.where(jnp.logical_and(open_, jnp.logical_not(pred)), mid, bhi))

    cnt, _ = lax.fori_loop(
        0, 15, _bstep, (jnp.int32(0), jnp.int32(_SPT)))
    cx_v[...] = jnp.full((16,), cnt, jnp.int32)
    pltpu.sync_copy(cx_v, cnts_sh.at[pl.ds(sid * 16, 16)])

    zero16 = jnp.zeros((16,), jnp.float32)

    def _zrow(r, c):
        def _zcol(j, cc):
            x_v[_NB - 1, r, pl.ds(j * 16, 16)] = zero16
            return cc
        return lax.fori_loop(0, _D // 16, _zcol, c)

    lax.fori_loop(0, _CH, _zrow, 0)

    for k in range(_ZPT // _CH):
        pltpu.sync_copy(x_v.at[_NB - 1], acc.at[pl.ds(sid * _ZPT + k * _CH, _CH), :])
    _zt = _ZPT % _CH
    pltpu.sync_copy(
        x_v.at[_NB - 1, pl.ds(0, _zt), :],
        acc.at[pl.ds(sid * _ZPT + _ZPT - _zt, _zt), :],
    )
    plsc.subcore_barrier()

    pltpu.sync_copy(cnts_sh, cxr_v)
    svec = cxr_v[pl.ds(0, 16)]
    for r in range(1, _NS):
        svec = svec + cxr_v[pl.ds(r * 16, 16)]
    s = jnp.minimum(jnp.maximum(svec[0], 0), _E)

    c0 = (s + _CH - 1) // _CH
    c1 = s // _CH
    m = jnp.where(cid == 0, c0, _NCHT - c1)
    base_c = jnp.where(cid == 0, 0, c1)
    lo = base_c + (m * sid) // _NS
    hi = base_c + (m * (sid + 1)) // _NS
    kstop = hi - lo

    def _start_load(c, b):
        pos = jnp.maximum(lo + jnp.minimum(c, kstop - 1), 0)
        pltpu.async_copy(
            idx_hbm.at[pl.ds(pos * _CH, _CH)], idx_v.at[b], sem_in.at[b])
        pltpu.async_copy(
            x_hbm.at[pl.ds(pos * _CH, _CH), :], x_v.at[b], sem_in.at[b])

    def _wait_load(b):
        pltpu.make_async_copy(
            idx_hbm.at[pl.ds(0, _CH)], idx_v.at[b], sem_in.at[b]).wait()
        pltpu.make_async_copy(
            x_hbm.at[pl.ds(0, _CH), :], x_v.at[b], sem_in.at[b]).wait()

    def _scatter_desc(b):
        return pltpu.make_async_copy(x_v.at[b], acc.at[idx_v.at[b]], sem_sc.at[b])

    base_row = cid * _HN

    def _consume(c, b):
        _wait_load(b)
        voff = jnp.full(
            (16,), base_row - jnp.where(c < kstop, 0, 2 * _HN), jnp.int32)
        for q in range(_CH // 16):
            v = idx_v[b, pl.ds(q * 16, 16)]
            loc = jnp.minimum(
                (v - voff).astype(jnp.uint32), jnp.uint32(_HN))
            idx_v[b, pl.ds(q * 16, 16)] = loc.astype(jnp.int32)
        pltpu.async_copy(x_v.at[b], acc.at[idx_v.at[b]], sem_sc.at[b], add=True)

    for b in range(_NB - 1):
        _start_load(b, b)

    for b in range(_NB):
        bp = (b + _NB - 1) % _NB
        if b >= 1:
            _scatter_desc(bp).wait()
        _start_load(b + _NB - 1, bp)
        _consume(b, b)

    def _group(j, carry):
        for b in range(_NB):
            c = j * _NB + b
            bp = (b + _NB - 1) % _NB
            _scatter_desc(bp).wait()
            _start_load(c + _NB - 1, bp)
            _consume(c, b)
        return carry

    n_groups = jnp.maximum((kstop + _NB - 1) // _NB, 1)
    lax.fori_loop(1, n_groups, _group, 0)

    _scatter_desc(_NB - 1).wait()
    for b in range(_NB - 1):
        _wait_load(b)
    plsc.subcore_barrier()

    pltpu.sync_copy(
        acc.at[pl.ds(sid * _OPT, _OPT), :],
        out_hbm.at[pl.ds(base_row + sid * _OPT, _OPT), :],
    )

    @pl.when(sid == _NS - 1)
    def _tail():
        pltpu.sync_copy(
            acc.at[pl.ds(_NS * _OPT, _HN - _NS * _OPT), :],
            out_hbm.at[pl.ds(base_row + _NS * _OPT, _HN - _NS * _OPT), :],
        )


_sc_scatter = functools.partial(
    pl.kernel,
    out_type=jax.ShapeDtypeStruct((_N, _D), jnp.float32),
    mesh=plsc.VectorSubcoreMesh(core_axis_name="c", subcore_axis_name="s"),
    scratch_types=[
        pltpu.VMEM((_NB, _CH), jnp.int32),
        pltpu.VMEM((_NB, _CH, _D), jnp.float32),
        pltpu.VMEM((_SPT + 16,), jnp.int32),
        pltpu.VMEM((16,), jnp.int32),
        pltpu.VMEM((_NS * 16,), jnp.int32),
        pltpu.VMEM_SHARED((_AN, _D), jnp.float32),
        pltpu.VMEM_SHARED((_NS * 16,), jnp.int32),
        pltpu.SemaphoreType.DMA((_NB,)),
        pltpu.SemaphoreType.DMA((_NB,)),
    ],
)(_sc_body)


def kernel(x, index, dim_size):
    del dim_size
    return _sc_scatter(x, index.astype(jnp.int32))


# speedup vs baseline: 1.0160x; 1.0101x over previous
import functools

import jax
import jax.numpy as jnp
from jax import lax
from jax.experimental import pallas as pl
from jax.experimental.pallas import tpu as pltpu
from jax.experimental.pallas import tpu_sc as plsc

_E = 320000
_D = 128
_N = 10000
_HN = _N // 2
_NC = 2
_NS = 16
_CH = 64
_NCHT = _E // _CH
_NB = 5
_AN = _HN + 8
_ZPT = _AN // _NS
_OPT = 312
_SPT = _E // _NS


def _sc_body(x_hbm, idx_hbm, out_hbm, idx_v, x_v, scan_v, cx_v, cxr_v, acc,
             cnts_sh, sem_in, sem_sc):
    cid = lax.axis_index("c")
    sid = lax.axis_index("s")

    pltpu.sync_copy(idx_hbm.at[pl.ds(sid * _SPT, _SPT)], scan_v.at[pl.ds(0, _SPT)])

    def _bstep(_, lohi):
        blo, bhi = lohi
        mid = (blo + bhi) // 2
        v = scan_v[pl.ds(mid, 16)]
        open_ = blo < bhi
        pred = jnp.logical_and(open_, v[0] < _HN)
        return (jnp.where(pred, mid + 1, blo),
                jnp.where(jnp.logical_and(open_, jnp.logical_not(pred)), mid, bhi))

    cnt, _ = lax.fori_loop(
        0, 15, _bstep, (jnp.int32(0), jnp.int32(_SPT)))
    cx_v[...] = jnp.full((16,), cnt, jnp.int32)
    pltpu.sync_copy(cx_v, cnts_sh.at[pl.ds(sid * 16, 16)])

    zero16 = jnp.zeros((16,), jnp.float32)

    def _zrow(r, c):
        def _zcol(j, cc):
            x_v[_NB - 1, r, pl.ds(j * 16, 16)] = zero16
            return cc
        return lax.fori_loop(0, _D // 16, _zcol, c)

    lax.fori_loop(0, _CH, _zrow, 0)

    for k in range(_ZPT // _CH):
        pltpu.sync_copy(x_v.at[_NB - 1], acc.at[pl.ds(sid * _ZPT + k * _CH, _CH), :])
    _zt = _ZPT % _CH
    pltpu.sync_copy(
        x_v.at[_NB - 1, pl.ds(0, _zt), :],
        acc.at[pl.ds(sid * _ZPT + _ZPT - _zt, _zt), :],
    )
    plsc.subcore_barrier()

    pltpu.sync_copy(cnts_sh, cxr_v)
    svec = cxr_v[pl.ds(0, 16)]
    for r in range(1, _NS):
        svec = svec + cxr_v[pl.ds(r * 16, 16)]
    s = jnp.minimum(jnp.maximum(svec[0], 0), _E)

    c0 = (s + _CH - 1) // _CH
    c1 = s // _CH
    m = jnp.where(cid == 0, c0, _NCHT - c1)
    base_c = jnp.where(cid == 0, 0, c1)
    lo = base_c + (m * sid) // _NS
    hi = base_c + (m * (sid + 1)) // _NS
    kstop = hi - lo

    def _start_load(c, b):
        pos = jnp.maximum(lo + jnp.minimum(c, kstop - 1), 0)
        pltpu.async_copy(
            idx_hbm.at[pl.ds(pos * _CH, _CH)], idx_v.at[b], sem_in.at[b])
        pltpu.async_copy(
            x_hbm.at[pl.ds(pos * _CH, _CH), :], x_v.at[b], sem_in.at[b])

    def _wait_load(b):
        pltpu.make_async_copy(
            idx_hbm.at[pl.ds(0, _CH)], idx_v.at[b], sem_in.at[b]).wait()
        pltpu.make_async_copy(
            x_hbm.at[pl.ds(0, _CH), :], x_v.at[b], sem_in.at[b]).wait()

    def _scatter_desc(b):
        return pltpu.make_async_copy(x_v.at[b], acc.at[idx_v.at[b]], sem_sc.at[b])

    base_row = cid * _HN

    def _consume(c, b):
        _wait_load(b)
        voff = jnp.full(
            (16,), base_row - jnp.where(c < kstop, 0, 2 * _HN), jnp.int32)
        for q in range(_CH // 16):
            v = idx_v[b, pl.ds(q * 16, 16)]
            loc = jnp.minimum(
                (v - voff).astype(jnp.uint32), jnp.uint32(_HN))
            idx_v[b, pl.ds(q * 16, 16)] = loc.astype(jnp.int32)
        pltpu.async_copy(x_v.at[b], acc.at[idx_v.at[b]], sem_sc.at[b], add=True)

    for b in range(_NB - 1):
        _start_load(b, b)

    for b in range(_NB):
        bp = (b + _NB - 1) % _NB
        if b >= 1:
            _scatter_desc(bp).wait()
        _start_load(b + _NB - 1, bp)
        _consume(b, b)

    def _group(j, carry):
        for b in range(_NB):
            c = j * _NB + b
            bp = (b + _NB - 1) % _NB
            _scatter_desc(bp).wait()
            _start_load(c + _NB - 1, bp)
            _consume(c, b)
        return carry

    n_groups = jnp.maximum((kstop + _NB - 1) // _NB, 1)
    lax.fori_loop(1, n_groups, _group, 0)

    _scatter_desc(_NB - 1).wait()
    for b in range(_NB - 1):
        _wait_load(b)
    plsc.subcore_barrier()

    pltpu.sync_copy(
        acc.at[pl.ds(sid * _OPT, _OPT), :],
        out_hbm.at[pl.ds(base_row + sid * _OPT, _OPT), :],
    )

    @pl.when(sid == _NS - 1)
    def _tail():
        pltpu.sync_copy(
            acc.at[pl.ds(_NS * _OPT, _HN - _NS * _OPT), :],
            out_hbm.at[pl.ds(base_row + _NS * _OPT, _HN - _NS * _OPT), :],
        )


_sc_scatter = functools.partial(
    pl.kernel,
    out_type=jax.ShapeDtypeStruct((_N, _D), jnp.float32),
    mesh=plsc.VectorSubcoreMesh(core_axis_name="c", subcore_axis_name="s"),
    scratch_types=[
        pltpu.VMEM((_NB, _CH), jnp.int32),
        pltpu.VMEM((_NB, _CH, _D), jnp.float32),
        pltpu.VMEM((_SPT + 16,), jnp.int32),
        pltpu.VMEM((16,), jnp.int32),
        pltpu.VMEM((_NS * 16,), jnp.int32),
        pltpu.VMEM_SHARED((_AN, _D), jnp.float32),
        pltpu.VMEM_SHARED((_NS * 16,), jnp.int32),
        pltpu.SemaphoreType.DMA((_NB,)),
        pltpu.SemaphoreType.DMA((_NB,)),
    ],
)(_sc_body)


def kernel(x, index, dim_size):
    del dim_size
    return _sc_scatter(x, index.astype(jnp.int32))
